# baseline (speedup 1.0000x reference)
import functools

import jax
import jax.numpy as jnp
from jax import lax
from jax.experimental import pallas as pl
from jax.experimental.pallas import tpu as pltpu
from jax.experimental.pallas import tpu_sc as plsc

B = 16384
L = 20
NC = 2
NS = 16
NW = NC * NS
BPW = B // NW
CH = 128

UID_CHUNKS = BPW // CH
SMALL_ROWS = 5 * B
SMALL_IDX_PW = SMALL_ROWS // NW // CH
LAB_ROWS = 2 * B * L
LAB_IDX_PW = LAB_ROWS // NW // CH
LAB_INNER = 16
LAB_OUTER = LAB_IDX_PW // LAB_INNER
SMALL_INNER = 10
SMALL_OUTER = SMALL_IDX_PW // SMALL_INNER


def _sc_gather_body(uid_idx, iid_idx, cat_idx, small_idx, lab_idx,
                    uid_tbl, iid_tbl, cat_tbl, small_tbl, lab_tbl,
                    uid_out, iid_out, cat_out, small_out, lab_out,
                    idxv, rows64, rows32, rowsc, rows16, sem):
  wid = lax.axis_index("s") * NC + lax.axis_index("c")

  def simple_gather(idx_hbm, tbl, out_hbm, rowsv, n_chunks):
    pltpu.sync_copy(idx_hbm.at[wid], idxv.at[pl.ds(0, n_chunks)])
    descs = []
    for j in range(n_chunks):
      descs.append(
          pltpu.async_copy(tbl.at[idxv.at[j]],
                           rowsv.at[pl.ds(j * CH, CH)], sem))
    for d in descs:
      d.wait()
    pltpu.sync_copy(rowsv.at[pl.ds(0, n_chunks * CH)],
                    out_hbm.at[pl.ds(wid * n_chunks * CH, n_chunks * CH)])

  simple_gather(uid_idx, uid_tbl, uid_out, rows64, UID_CHUNKS)
  simple_gather(iid_idx, iid_tbl, iid_out, rows64, UID_CHUNKS)
  simple_gather(cat_idx, cat_tbl, cat_out, rowsc, UID_CHUNKS)

  pltpu.sync_copy(small_idx.at[wid], idxv.at[pl.ds(0, SMALL_IDX_PW)])

  def small_chunk(c, carry):
    descs = []
    for j in range(SMALL_INNER):
      descs.append(
          pltpu.async_copy(small_tbl.at[idxv.at[c * SMALL_INNER + j]],
                           rows16.at[pl.ds(j * CH, CH)], sem))
    for d in descs:
      d.wait()
    pltpu.sync_copy(
        rows16,
        small_out.at[pl.ds(wid * SMALL_IDX_PW * CH + c * SMALL_INNER * CH,
                           SMALL_INNER * CH)])
    return carry

  lax.fori_loop(0, SMALL_OUTER, small_chunk, 0)

  def lab_chunk(c, carry):
    pltpu.sync_copy(lab_idx.at[wid, pl.ds(c * LAB_INNER, LAB_INNER)],
                    idxv.at[pl.ds(0, LAB_INNER)])
    descs = []
    for j in range(LAB_INNER):
      descs.append(
          pltpu.async_copy(lab_tbl.at[idxv.at[j]],
                           rows32.at[pl.ds(j * CH, CH)], sem))
    for d in descs:
      d.wait()
    pltpu.sync_copy(
        rows32,
        lab_out.at[pl.ds(wid * LAB_IDX_PW * CH + c * LAB_INNER * CH,
                         LAB_INNER * CH)])
    return carry

  lax.fori_loop(0, LAB_OUTER, lab_chunk, 0)


def _sc_gather(uid_idx, iid_idx, cat_idx, small_idx, lab_idx,
               uid_tbl, iid_tbl, cat_tbl, small_tbl, lab_tbl):
  mesh = plsc.VectorSubcoreMesh(core_axis_name="c", subcore_axis_name="s",
                                num_cores=NC, num_subcores=NS)
  f = pl.kernel(
      _sc_gather_body,
      out_type=(
          jax.ShapeDtypeStruct((B, 64), jnp.bfloat16),
          jax.ShapeDtypeStruct((B, 64), jnp.bfloat16),
          jax.ShapeDtypeStruct((B, 32), jnp.float32),
          jax.ShapeDtypeStruct((SMALL_ROWS, 16), jnp.float32),
          jax.ShapeDtypeStruct((LAB_ROWS, 32), jnp.bfloat16),
      ),
      mesh=mesh,
      compiler_params=pltpu.CompilerParams(use_tc_tiling_on_sc=False),
      scratch_types=[
          pltpu.VMEM((SMALL_IDX_PW, CH), jnp.int32),
          pltpu.VMEM((UID_CHUNKS * CH, 64), jnp.bfloat16),
          pltpu.VMEM((LAB_INNER * CH, 32), jnp.bfloat16),
          pltpu.VMEM((UID_CHUNKS * CH, 32), jnp.float32),
          pltpu.VMEM((SMALL_INNER * CH, 16), jnp.float32),
          pltpu.SemaphoreType.DMA,
      ],
  )
  return f(uid_idx, iid_idx, cat_idx, small_idx, lab_idx,
           uid_tbl, iid_tbl, cat_tbl, small_tbl, lab_tbl)


BS = 512


def _tc_dense_body(uid_ref, iid_ref, cat_ref, small_ref, lab_ref, w_ref,
                   u1_ref, ub1_ref, u2_ref, ub2_ref,
                   i1_ref, ib1_ref, i2_ref, ib2_ref, out_ref):
  w = w_ref[0]

  def pool(labs16):
    labs = labs16.astype(jnp.float32)
    s = jnp.sum(labs * w[None, None, :], axis=2)
    m = jnp.max(s, axis=1, keepdims=True)
    e = jnp.exp(s - m)
    wt = e / jnp.sum(e, axis=1, keepdims=True)
    return jnp.sum(labs * wt[:, :, None], axis=1)

  u_pool = pool(lab_ref[0])
  i_pool = pool(lab_ref[1])

  user_feat = jnp.concatenate(
      [uid_ref[...].astype(jnp.float32), small_ref[0], small_ref[1],
       small_ref[2], small_ref[3], u_pool], axis=1)
  item_feat = jnp.concatenate(
      [iid_ref[...].astype(jnp.float32), cat_ref[...], small_ref[4], i_pool],
      axis=1)

  hu = jnp.maximum(
      jnp.dot(user_feat, u1_ref[...], preferred_element_type=jnp.float32)
      + ub1_ref[0], 0.0)
  uvec = jnp.dot(hu, u2_ref[...], preferred_element_type=jnp.float32) \
      + ub2_ref[0]
  hi = jnp.dot(item_feat, i1_ref[...], preferred_element_type=jnp.float32) \
      + ib1_ref[0]
  ivec = jnp.dot(hi, i2_ref[...], preferred_element_type=jnp.float32) \
      + ib2_ref[0]
  logit = jnp.sum(uvec * ivec, axis=1, keepdims=True)
  out_ref[...] = 1.0 / (1.0 + jnp.exp(-logit))


def _tc_dense(uid_emb, iid_emb, cat_emb, small_emb, lab_emb, w_pool,
              U1, Ub1, U2, Ub2, I1, Ib1, I2, Ib2):
  grid = (B // BS,)
  full = lambda shape: pl.BlockSpec(shape, lambda i: tuple(0 for _ in shape))
  out = pl.pallas_call(
      _tc_dense_body,
      grid=grid,
      in_specs=[
          pl.BlockSpec((BS, 64), lambda i: (i, 0)),
          pl.BlockSpec((BS, 64), lambda i: (i, 0)),
          pl.BlockSpec((BS, 32), lambda i: (i, 0)),
          pl.BlockSpec((5, BS, 16), lambda i: (0, i, 0)),
          pl.BlockSpec((2, BS, L, 32), lambda i: (0, i, 0, 0)),
          full((1, 32)),
          full((160, 256)), full((1, 256)), full((256, 128)), full((1, 128)),
          full((144, 256)), full((1, 256)), full((256, 128)), full((1, 128)),
      ],
      out_specs=pl.BlockSpec((BS, 1), lambda i: (i, 0)),
      out_shape=jax.ShapeDtypeStruct((B, 1), jnp.float32),
  )(uid_emb, iid_emb, cat_emb, small_emb, lab_emb, w_pool,
    U1, Ub1, U2, Ub2, I1, Ib1, I2, Ib2)
  return out


def kernel(user_id, gender_id, job_id, user_city_id, age_bucket, user_labels,
           item_id, category_id, item_city_id, item_labels,
           user_id_table, gender_table, job_table, city_table, age_table,
           item_id_table, category_table, label_table, w_pool,
           U1, Ub1, U2, Ub2, I1, Ib1, I2, Ib2):
  i32 = jnp.int32
  small_tbl = jnp.concatenate(
      [gender_table, job_table, city_table, age_table], axis=0)
  small_idx = jnp.concatenate([
      gender_id.astype(i32),
      job_id.astype(i32) + 3,
      user_city_id.astype(i32) + 104,
      age_bucket.astype(i32) + 1105,
      item_city_id.astype(i32) + 104,
  ]).reshape(NW, SMALL_IDX_PW, CH)
  lab_idx = jnp.concatenate(
      [user_labels.reshape(-1).astype(i32),
       item_labels.reshape(-1).astype(i32)]).reshape(NW, LAB_IDX_PW, CH)

  uid_emb, iid_emb, cat_emb, small_emb, lab_emb = _sc_gather(
      user_id.astype(i32).reshape(NW, UID_CHUNKS, CH),
      item_id.astype(i32).reshape(NW, UID_CHUNKS, CH),
      category_id.astype(i32).reshape(NW, UID_CHUNKS, CH),
      small_idx, lab_idx,
      user_id_table.astype(jnp.bfloat16), item_id_table.astype(jnp.bfloat16),
      category_table, small_tbl, label_table.astype(jnp.bfloat16))

  out = _tc_dense(uid_emb, iid_emb, cat_emb,
                  small_emb.reshape(5, B, 16),
                  lab_emb.reshape(2, B, L, 32),
                  w_pool.reshape(1, 32),
                  U1, Ub1.reshape(1, 256), U2, Ub2.reshape(1, 128),
                  I1, Ib1.reshape(1, 256), I2, Ib2.reshape(1, 128))
  return out.reshape(B)


# speedup vs baseline: 1.2262x; 1.2262x over previous
import functools

import jax
import jax.numpy as jnp
from jax import lax
from jax.experimental import pallas as pl
from jax.experimental.pallas import tpu as pltpu
from jax.experimental.pallas import tpu_sc as plsc

B = 16384
L = 20
NC = 2
NS = 16
NW = NC * NS
BPW = B // NW
CH = 128

ID_CHUNKS = BPW // CH
SMALL_ROWS = 5 * B
SMALL_IDX_PW = SMALL_ROWS // NW // CH
LAB_ROWS = 2 * B * L
LAB_IDX_PW = LAB_ROWS // NW // CH
LAB_INNER = 16
LAB_OUTER = LAB_IDX_PW // LAB_INNER
SMALL_INNER = 10
SMALL_OUTER = SMALL_IDX_PW // SMALL_INNER


def _mesh():
  return plsc.VectorSubcoreMesh(core_axis_name="c", subcore_axis_name="s",
                                num_cores=NC, num_subcores=NS)


def _wid():
  return lax.axis_index("s") * NC + lax.axis_index("c")


def _sc_misc_body(cat_idx, small_idx, lab_idx, cat_tbl, small_tbl, lab_tbl,
                  cat_out, small_out, lab_out,
                  idxv, rows32, rowsc, rows16, sem):
  wid = _wid()

  pltpu.sync_copy(cat_idx.at[wid], idxv.at[pl.ds(0, ID_CHUNKS)])
  descs = []
  for j in range(ID_CHUNKS):
    descs.append(
        pltpu.async_copy(cat_tbl.at[idxv.at[j]],
                         rowsc.at[pl.ds(j * CH, CH)], sem))
  for d in descs:
    d.wait()
  pltpu.sync_copy(rowsc, cat_out.at[pl.ds(wid * ID_CHUNKS * CH,
                                          ID_CHUNKS * CH)])

  pltpu.sync_copy(small_idx.at[wid], idxv.at[pl.ds(0, SMALL_IDX_PW)])

  def small_chunk(c, carry):
    descs = []
    for j in range(SMALL_INNER):
      descs.append(
          pltpu.async_copy(small_tbl.at[idxv.at[c * SMALL_INNER + j]],
                           rows16.at[pl.ds(j * CH, CH)], sem))
    for d in descs:
      d.wait()
    pltpu.sync_copy(
        rows16,
        small_out.at[pl.ds(wid * SMALL_IDX_PW * CH + c * SMALL_INNER * CH,
                           SMALL_INNER * CH)])
    return carry

  lax.fori_loop(0, SMALL_OUTER, small_chunk, 0)

  def lab_chunk(c, carry):
    pltpu.sync_copy(lab_idx.at[wid, pl.ds(c * LAB_INNER, LAB_INNER)],
                    idxv.at[pl.ds(0, LAB_INNER)])
    descs = []
    for j in range(LAB_INNER):
      descs.append(
          pltpu.async_copy(lab_tbl.at[idxv.at[j]],
                           rows32.at[pl.ds(j * CH, CH)], sem))
    for d in descs:
      d.wait()
    pltpu.sync_copy(
        rows32,
        lab_out.at[pl.ds(wid * LAB_IDX_PW * CH + c * LAB_INNER * CH,
                         LAB_INNER * CH)])
    return carry

  lax.fori_loop(0, LAB_OUTER, lab_chunk, 0)


def _sc_misc(cat_idx, small_idx, lab_idx, cat_tbl, small_tbl, lab_tbl):
  f = pl.kernel(
      _sc_misc_body,
      out_type=(
          jax.ShapeDtypeStruct((B, 32), jnp.float32),
          jax.ShapeDtypeStruct((SMALL_ROWS, 16), jnp.float32),
          jax.ShapeDtypeStruct((LAB_ROWS, 32), jnp.bfloat16),
      ),
      mesh=_mesh(),
      compiler_params=pltpu.CompilerParams(use_tc_tiling_on_sc=False),
      scratch_types=[
          pltpu.VMEM((SMALL_IDX_PW, CH), jnp.int32),
          pltpu.VMEM((LAB_INNER * CH, 32), jnp.bfloat16),
          pltpu.VMEM((ID_CHUNKS * CH, 32), jnp.float32),
          pltpu.VMEM((SMALL_INNER * CH, 16), jnp.float32),
          pltpu.SemaphoreType.DMA,
      ],
  )
  return f(cat_idx, small_idx, lab_idx, cat_tbl, small_tbl, lab_tbl)


def _sc_ids_body(uq_idx, iq_idx, uid_tbl, iid_tbl, uid_out, iid_out,
                 idxv, rows2d, sem):
  wid = _wid()

  def id_gather(idx_hbm, tbl, out_hbm):
    pltpu.sync_copy(idx_hbm.at[wid], idxv.at[pl.ds(0, ID_CHUNKS)])
    descs = []
    for j in range(ID_CHUNKS):
      descs.append(
          pltpu.async_copy(tbl.at[idxv.at[j]],
                           rows2d.at[pl.ds(j * CH, CH)], sem))
    for d in descs:
      d.wait()
    pltpu.sync_copy(rows2d, out_hbm.at[pl.ds(wid * BPW, BPW)])

  id_gather(uq_idx, uid_tbl, uid_out)
  id_gather(iq_idx, iid_tbl, iid_out)


def _sc_ids(uq_idx, iq_idx, uid_tbl3, iid_tbl3):
  f = pl.kernel(
      _sc_ids_body,
      out_type=(
          jax.ShapeDtypeStruct((B, 128), jnp.float32),
          jax.ShapeDtypeStruct((B, 128), jnp.float32),
      ),
      mesh=_mesh(),
      compiler_params=pltpu.CompilerParams(use_tc_tiling_on_sc=True),
      scratch_types=[
          pltpu.VMEM((8, CH), jnp.int32),
          pltpu.VMEM((BPW, 128), jnp.float32),
          pltpu.SemaphoreType.DMA,
      ],
  )
  return f(uq_idx, iq_idx, uid_tbl3, iid_tbl3)


BS = 512


def _tc_dense_body(uid_ref, uq_ref, iid_ref, iq_ref, cat_ref, small_ref,
                   lab_ref, w_ref,
                   u1_ref, ub1_ref, u2_ref, ub2_ref,
                   i1_ref, ib1_ref, i2_ref, ib2_ref, out_ref):
  w = w_ref[0]

  def pick_half(rows_ref, q_ref):
    rows = rows_ref[...]
    q = q_ref[...]
    return jnp.where(q == 0, rows[:, :64], rows[:, 64:])

  uid_emb = pick_half(uid_ref, uq_ref)
  iid_emb = pick_half(iid_ref, iq_ref)

  def pool(labs16):
    labs = labs16.astype(jnp.float32)
    s = jnp.sum(labs * w[None, None, :], axis=2)
    m = jnp.max(s, axis=1, keepdims=True)
    e = jnp.exp(s - m)
    wt = e / jnp.sum(e, axis=1, keepdims=True)
    return jnp.sum(labs * wt[:, :, None], axis=1)

  u_pool = pool(lab_ref[0])
  i_pool = pool(lab_ref[1])

  user_feat = jnp.concatenate(
      [uid_emb, small_ref[0], small_ref[1], small_ref[2], small_ref[3],
       u_pool], axis=1)
  item_feat = jnp.concatenate(
      [iid_emb, cat_ref[...], small_ref[4], i_pool], axis=1)

  hu = jnp.maximum(
      jnp.dot(user_feat, u1_ref[...], preferred_element_type=jnp.float32)
      + ub1_ref[0], 0.0)
  uvec = jnp.dot(hu, u2_ref[...], preferred_element_type=jnp.float32) \
      + ub2_ref[0]
  hi = jnp.dot(item_feat, i1_ref[...], preferred_element_type=jnp.float32) \
      + ib1_ref[0]
  ivec = jnp.dot(hi, i2_ref[...], preferred_element_type=jnp.float32) \
      + ib2_ref[0]
  logit = jnp.sum(uvec * ivec, axis=1, keepdims=True)
  out_ref[...] = 1.0 / (1.0 + jnp.exp(-logit))


def _tc_dense(uid_emb, uq, iid_emb, iq, cat_emb, small_emb, lab_emb, w_pool,
              U1, Ub1, U2, Ub2, I1, Ib1, I2, Ib2):
  grid = (B // BS,)
  full = lambda shape: pl.BlockSpec(shape, lambda i: tuple(0 for _ in shape))
  out = pl.pallas_call(
      _tc_dense_body,
      grid=grid,
      in_specs=[
          pl.BlockSpec((BS, 128), lambda i: (i, 0)),
          pl.BlockSpec((BS, 1), lambda i: (i, 0)),
          pl.BlockSpec((BS, 128), lambda i: (i, 0)),
          pl.BlockSpec((BS, 1), lambda i: (i, 0)),
          pl.BlockSpec((BS, 32), lambda i: (i, 0)),
          pl.BlockSpec((5, BS, 16), lambda i: (0, i, 0)),
          pl.BlockSpec((2, BS, L, 32), lambda i: (0, i, 0, 0)),
          full((1, 32)),
          full((160, 256)), full((1, 256)), full((256, 128)), full((1, 128)),
          full((144, 256)), full((1, 256)), full((256, 128)), full((1, 128)),
      ],
      out_specs=pl.BlockSpec((BS, 1), lambda i: (i, 0)),
      out_shape=jax.ShapeDtypeStruct((B, 1), jnp.float32),
  )(uid_emb, uq, iid_emb, iq, cat_emb, small_emb, lab_emb, w_pool,
    U1, Ub1, U2, Ub2, I1, Ib1, I2, Ib2)
  return out


def kernel(user_id, gender_id, job_id, user_city_id, age_bucket, user_labels,
           item_id, category_id, item_city_id, item_labels,
           user_id_table, gender_table, job_table, city_table, age_table,
           item_id_table, category_table, label_table, w_pool,
           U1, Ub1, U2, Ub2, I1, Ib1, I2, Ib2):
  i32 = jnp.int32
  bf16 = jnp.bfloat16
  small_tbl = jnp.concatenate(
      [gender_table, job_table, city_table, age_table], axis=0)
  small_idx = jnp.concatenate([
      gender_id.astype(i32),
      job_id.astype(i32) + 3,
      user_city_id.astype(i32) + 104,
      age_bucket.astype(i32) + 1105,
      item_city_id.astype(i32) + 104,
  ]).reshape(NW, SMALL_IDX_PW, CH)
  lab_idx = jnp.concatenate(
      [user_labels.reshape(-1).astype(i32),
       item_labels.reshape(-1).astype(i32)]).reshape(NW, LAB_IDX_PW, CH)

  cat_emb, small_emb, lab_emb = _sc_misc(
      category_id.astype(i32).reshape(NW, ID_CHUNKS, CH),
      small_idx, lab_idx,
      category_table, small_tbl, label_table.astype(bf16))

  uid2 = user_id_table.reshape(500000, 128)
  iid2 = item_id_table.reshape(500000, 128)
  uid_q = user_id.astype(i32)
  iid_q = item_id.astype(i32)
  uid_emb, iid_emb = _sc_ids(
      (uid_q // 2).reshape(NW, ID_CHUNKS, CH),
      (iid_q // 2).reshape(NW, ID_CHUNKS, CH),
      uid2, iid2)

  out = _tc_dense(uid_emb, (uid_q % 2).reshape(B, 1),
                  iid_emb, (iid_q % 2).reshape(B, 1),
                  cat_emb,
                  small_emb.reshape(5, B, 16),
                  lab_emb.reshape(2, B, L, 32),
                  w_pool.reshape(1, 32),
                  U1, Ub1.reshape(1, 256), U2, Ub2.reshape(1, 128),
                  I1, Ib1.reshape(1, 256), I2, Ib2.reshape(1, 128))
  return out.reshape(B)


# speedup vs baseline: 1.4930x; 1.2176x over previous
import functools

import jax
import jax.numpy as jnp
from jax import lax
from jax.experimental import pallas as pl
from jax.experimental.pallas import tpu as pltpu
from jax.experimental.pallas import tpu_sc as plsc

B = 16384
L = 20
NC = 2
NS = 16
NW = NC * NS
BPW = B // NW
CH = 128

ID_CHUNKS = BPW // CH
SMALL_ROWS = 5 * B
SMALL_IDX_PW = SMALL_ROWS // NW // CH
LAB_ROWS = 2 * B * L
LAB_IDX_PW = LAB_ROWS // NW // CH
LAB_INNER = 16
LAB_OUTER = LAB_IDX_PW // LAB_INNER
SMALL_INNER = 10
SMALL_OUTER = SMALL_IDX_PW // SMALL_INNER


def _mesh():
  return plsc.VectorSubcoreMesh(core_axis_name="c", subcore_axis_name="s",
                                num_cores=NC, num_subcores=NS)


def _wid():
  return lax.axis_index("s") * NC + lax.axis_index("c")


def _sc_gather_body(uid_idx, iid_idx, cat_idx, small_idx, lab_idx,
                    uid_tbl, iid_tbl, cat_tbl, small_tbl, lab_tbl,
                    uid_out, iid_out, cat_out, small_out, lab_out,
                    idxv, rows64, rows32, rowsc, rows16, sem):
  wid = _wid()

  def rows_gather(idx_hbm, tbl, out_hbm, rowsv):
    pltpu.sync_copy(idx_hbm.at[wid], idxv.at[pl.ds(0, ID_CHUNKS)])
    descs = []
    for j in range(ID_CHUNKS):
      descs.append(
          pltpu.async_copy(tbl.at[idxv.at[j]],
                           rowsv.at[pl.ds(j * CH, CH)], sem))
    for d in descs:
      d.wait()
    pltpu.sync_copy(rowsv, out_hbm.at[pl.ds(wid * ID_CHUNKS * CH,
                                            ID_CHUNKS * CH)])

  rows_gather(uid_idx, uid_tbl, uid_out, rows64)
  rows_gather(iid_idx, iid_tbl, iid_out, rows64)
  rows_gather(cat_idx, cat_tbl, cat_out, rowsc)

  pltpu.sync_copy(small_idx.at[wid], idxv.at[pl.ds(0, SMALL_IDX_PW)])

  def small_chunk(c, carry):
    descs = []
    for j in range(SMALL_INNER):
      descs.append(
          pltpu.async_copy(small_tbl.at[idxv.at[c * SMALL_INNER + j]],
                           rows16.at[pl.ds(j * CH, CH)], sem))
    for d in descs:
      d.wait()
    pltpu.sync_copy(
        rows16,
        small_out.at[pl.ds(wid * SMALL_IDX_PW * CH + c * SMALL_INNER * CH,
                           SMALL_INNER * CH)])
    return carry

  lax.fori_loop(0, SMALL_OUTER, small_chunk, 0)

  def lab_chunk(c, carry):
    pltpu.sync_copy(lab_idx.at[wid, pl.ds(c * LAB_INNER, LAB_INNER)],
                    idxv.at[pl.ds(0, LAB_INNER)])
    descs = []
    for j in range(LAB_INNER):
      descs.append(
          pltpu.async_copy(lab_tbl.at[idxv.at[j]],
                           rows32.at[pl.ds(j * CH, CH)], sem))
    for d in descs:
      d.wait()
    pltpu.sync_copy(
        rows32,
        lab_out.at[pl.ds(wid * LAB_IDX_PW * CH + c * LAB_INNER * CH,
                         LAB_INNER * CH)])
    return carry

  lax.fori_loop(0, LAB_OUTER, lab_chunk, 0)


def _sc_gather(uid_idx, iid_idx, cat_idx, small_idx, lab_idx,
               uid_tbl, iid_tbl, cat_tbl, small_tbl, lab_tbl):
  f = pl.kernel(
      _sc_gather_body,
      out_type=(
          jax.ShapeDtypeStruct((B, 64), jnp.float32),
          jax.ShapeDtypeStruct((B, 64), jnp.float32),
          jax.ShapeDtypeStruct((B, 32), jnp.float32),
          jax.ShapeDtypeStruct((SMALL_ROWS, 16), jnp.float32),
          jax.ShapeDtypeStruct((LAB_ROWS, 32), jnp.bfloat16),
      ),
      mesh=_mesh(),
      compiler_params=pltpu.CompilerParams(use_tc_tiling_on_sc=False),
      scratch_types=[
          pltpu.VMEM((SMALL_IDX_PW, CH), jnp.int32),
          pltpu.VMEM((ID_CHUNKS * CH, 64), jnp.float32),
          pltpu.VMEM((LAB_INNER * CH, 32), jnp.bfloat16),
          pltpu.VMEM((ID_CHUNKS * CH, 32), jnp.float32),
          pltpu.VMEM((SMALL_INNER * CH, 16), jnp.float32),
          pltpu.SemaphoreType.DMA,
      ],
  )
  return f(uid_idx, iid_idx, cat_idx, small_idx, lab_idx,
           uid_tbl, iid_tbl, cat_tbl, small_tbl, lab_tbl)


BS = 512


def _tc_dense_body(uid_ref, iid_ref, cat_ref, small_ref,
                   lab_ref, w20_ref, e20_ref, p32_ref,
                   u1_ref, ub1_ref, u2_ref, ub2_ref,
                   i1_ref, ib1_ref, i2_ref, ib2_ref, out_ref):
  uid_emb = uid_ref[...]
  iid_emb = iid_ref[...]

  w20 = w20_ref[...]
  e20 = e20_ref[...]
  p32 = p32_ref[...]

  def pool(x16):
    x = x16.astype(jnp.float32)
    s = jnp.dot(x, w20, preferred_element_type=jnp.float32)
    m = jnp.max(s, axis=1, keepdims=True)
    e = jnp.exp(s - m)
    wt = e / jnp.sum(e, axis=1, keepdims=True)
    wt640 = jnp.dot(wt, e20, preferred_element_type=jnp.float32)
    return jnp.dot(x * wt640, p32, preferred_element_type=jnp.float32)

  u_pool = pool(lab_ref[0])
  i_pool = pool(lab_ref[1])

  user_feat = jnp.concatenate(
      [uid_emb, small_ref[0], small_ref[1], small_ref[2], small_ref[3],
       u_pool], axis=1)
  item_feat = jnp.concatenate(
      [iid_emb, cat_ref[...], small_ref[4], i_pool], axis=1)

  hu = jnp.maximum(
      jnp.dot(user_feat, u1_ref[...], preferred_element_type=jnp.float32)
      + ub1_ref[0], 0.0)
  uvec = jnp.dot(hu, u2_ref[...], preferred_element_type=jnp.float32) \
      + ub2_ref[0]
  hi = jnp.dot(item_feat, i1_ref[...], preferred_element_type=jnp.float32) \
      + ib1_ref[0]
  ivec = jnp.dot(hi, i2_ref[...], preferred_element_type=jnp.float32) \
      + ib2_ref[0]
  logit = jnp.sum(uvec * ivec, axis=1, keepdims=True)
  out_ref[...] = 1.0 / (1.0 + jnp.exp(-logit))


def _tc_dense(uid_emb, iid_emb, cat_emb, small_emb, lab_emb, w20, e20, p32,
              U1, Ub1, U2, Ub2, I1, Ib1, I2, Ib2):
  grid = (B // BS,)
  full = lambda shape: pl.BlockSpec(shape, lambda i: tuple(0 for _ in shape))
  out = pl.pallas_call(
      _tc_dense_body,
      grid=grid,
      in_specs=[
          pl.BlockSpec((BS, 64), lambda i: (i, 0)),
          pl.BlockSpec((BS, 64), lambda i: (i, 0)),
          pl.BlockSpec((BS, 32), lambda i: (i, 0)),
          pl.BlockSpec((5, BS, 16), lambda i: (0, i, 0)),
          pl.BlockSpec((2, BS, 640), lambda i: (0, i, 0)),
          full((640, 20)), full((20, 640)), full((640, 32)),
          full((160, 256)), full((1, 256)), full((256, 128)), full((1, 128)),
          full((144, 256)), full((1, 256)), full((256, 128)), full((1, 128)),
      ],
      out_specs=pl.BlockSpec((BS, 1), lambda i: (i, 0)),
      out_shape=jax.ShapeDtypeStruct((B, 1), jnp.float32),
  )(uid_emb, iid_emb, cat_emb, small_emb, lab_emb, w20, e20, p32,
    U1, Ub1, U2, Ub2, I1, Ib1, I2, Ib2)
  return out


def kernel(user_id, gender_id, job_id, user_city_id, age_bucket, user_labels,
           item_id, category_id, item_city_id, item_labels,
           user_id_table, gender_table, job_table, city_table, age_table,
           item_id_table, category_table, label_table, w_pool,
           U1, Ub1, U2, Ub2, I1, Ib1, I2, Ib2):
  i32 = jnp.int32
  bf16 = jnp.bfloat16
  small_tbl = jnp.concatenate(
      [gender_table, job_table, city_table, age_table], axis=0)
  small_idx = jnp.concatenate([
      gender_id.astype(i32),
      job_id.astype(i32) + 3,
      user_city_id.astype(i32) + 104,
      age_bucket.astype(i32) + 1105,
      item_city_id.astype(i32) + 104,
  ]).reshape(NW, SMALL_IDX_PW, CH)
  lab_idx = jnp.concatenate(
      [user_labels.reshape(-1).astype(i32),
       item_labels.reshape(-1).astype(i32)]).reshape(NW, LAB_IDX_PW, CH)

  uid_emb, iid_emb, cat_emb, small_emb, lab_emb = _sc_gather(
      user_id.astype(i32).reshape(NW, ID_CHUNKS, CH),
      item_id.astype(i32).reshape(NW, ID_CHUNKS, CH),
      category_id.astype(i32).reshape(NW, ID_CHUNKS, CH),
      small_idx, lab_idx,
      user_id_table, item_id_table, category_table, small_tbl,
      label_table.astype(bf16))

  eye20 = jnp.eye(20, dtype=jnp.float32)
  w20 = jnp.kron(eye20, w_pool.reshape(32, 1))
  e20 = jnp.kron(eye20, jnp.ones((1, 32), jnp.float32))
  p32 = jnp.kron(jnp.ones((20, 1), jnp.float32),
                 jnp.eye(32, dtype=jnp.float32))

  out = _tc_dense(uid_emb, iid_emb, cat_emb,
                  small_emb.reshape(5, B, 16),
                  lab_emb.reshape(2, B, 640),
                  w20, e20, p32,
                  U1, Ub1.reshape(1, 256), U2, Ub2.reshape(1, 128),
                  I1, Ib1.reshape(1, 256), I2, Ib2.reshape(1, 128))
  return out.reshape(B)
